# baseline (speedup 1.0000x reference)
import functools

import jax
import jax.numpy as jnp
from jax import lax
from jax.experimental import pallas as pl
from jax.experimental.pallas import tpu as pltpu
from jax.experimental.pallas import tpu_sc as plsc

_NC = 2
_NS = 16
_NW = _NC * _NS
_CHUNK = 128
_NBUF = 6
_DIST = 4
_DIAG = 0


def _sc_body(n_chunks, table_hbm, idx_hbm, out_hbm, idx_v, rows_v, *sems):
    gsems = sems[:_NBUF]
    wsems = sems[_NBUF:]
    bpw = n_chunks * _CHUNK
    wid = lax.axis_index("s") * _NC + lax.axis_index("c")
    base = wid * bpw
    pltpu.sync_copy(idx_hbm.at[pl.ds(base, bpw)], idx_v)

    def start_gather(c, b):
        if _DIAG == 2:
            return
        pltpu.async_copy(
            table_hbm.at[idx_v.at[pl.ds(c * _CHUNK, _CHUNK)]], rows_v.at[b],
            gsems[b])

    def wait_gather(b):
        if _DIAG == 2:
            return
        pltpu.make_async_copy(
            table_hbm.at[idx_v.at[pl.ds(0, _CHUNK)]], rows_v.at[b],
            gsems[b]).wait()

    def start_write(c, b):
        if _DIAG == 1:
            return
        pltpu.async_copy(
            rows_v.at[b], out_hbm.at[pl.ds(base + c * _CHUNK, _CHUNK)],
            wsems[b])

    def wait_write(b):
        if _DIAG == 1:
            return
        pltpu.make_async_copy(
            rows_v.at[b], out_hbm.at[pl.ds(base, _CHUNK)], wsems[b]).wait()

    def turn(c, rot, wwait, prefetch):
        if prefetch:
            b2 = (rot + _DIST) % _NBUF
            if wwait:
                wait_write(b2)
            start_gather(c + _DIST, b2)
        b = rot % _NBUF
        wait_gather(b)
        start_write(c, b)

    for c in range(_DIST):
        start_gather(c, c)

    head = _NBUF - _DIST
    for c in range(head):
        turn(c, c, wwait=False, prefetch=True)

    n_tail = _DIST
    steady = n_chunks - head - n_tail
    n_groups = steady // _NBUF
    rem = steady - n_groups * _NBUF

    def group(g, carry):
        c0 = head + g * _NBUF
        for j in range(_NBUF):
            turn(c0 + j, head + j, wwait=True, prefetch=True)
        return carry

    lax.fori_loop(0, n_groups, group, 0)

    for j in range(rem):
        c = head + n_groups * _NBUF + j
        turn(c, head + j, wwait=True, prefetch=True)

    for c in range(n_chunks - n_tail, n_chunks):
        turn(c, c, wwait=False, prefetch=False)

    for c in range(n_chunks - _NBUF, n_chunks):
        wait_write(c % _NBUF)


@jax.jit
def _gather(table, idx):
    n, d = idx.shape[0], table.shape[1]
    assert n % (_NW * _CHUNK) == 0
    bpw = n // _NW
    n_chunks = bpw // _CHUNK
    assert n_chunks >= 2 * _NBUF
    mesh = plsc.VectorSubcoreMesh(core_axis_name="c", subcore_axis_name="s")
    f = pl.kernel(
        functools.partial(_sc_body, n_chunks),
        out_type=jax.ShapeDtypeStruct((n, d), jnp.float32),
        mesh=mesh,
        scratch_types=(
            [pltpu.VMEM((bpw,), jnp.int32),
             pltpu.VMEM((_NBUF, _CHUNK, d), jnp.float32)]
            + [pltpu.SemaphoreType.DMA] * (2 * _NBUF)
        ),
    )
    return f(table, idx)


def kernel(token_ids, embedding):
    b, h = token_ids.shape
    d = embedding.shape[1]
    idx = token_ids.T.reshape(-1).astype(jnp.int32)
    out = _gather(embedding, idx)
    return out.reshape(h, b, d).transpose(1, 0, 2)


# speedup vs baseline: 1.0013x; 1.0013x over previous
import functools

import jax
import jax.numpy as jnp
from jax import lax
from jax.experimental import pallas as pl
from jax.experimental.pallas import tpu as pltpu
from jax.experimental.pallas import tpu_sc as plsc

_NC = 2
_NS = 16
_NW = _NC * _NS
_CHUNK = 128
_NBUF = 6
_DIST = 4


def _sc_body(n_chunks, table_hbm, idx_hbm, out_hbm, idx_v, rows_v, *sems):
    gsems = sems[:_NBUF]
    wsems = sems[_NBUF:]
    bpw = n_chunks * _CHUNK
    wid = lax.axis_index("s") * _NC + lax.axis_index("c")
    base = wid * bpw
    pltpu.sync_copy(idx_hbm.at[pl.ds(base, bpw)], idx_v)

    def start_gather(c, b):
        pltpu.async_copy(
            table_hbm.at[idx_v.at[pl.ds(c * _CHUNK, _CHUNK)]], rows_v.at[b],
            gsems[b])

    def wait_gather(b):
        pltpu.make_async_copy(
            table_hbm.at[idx_v.at[pl.ds(0, _CHUNK)]], rows_v.at[b],
            gsems[b]).wait()

    def start_write(c, b):
        pltpu.async_copy(
            rows_v.at[b], out_hbm.at[pl.ds(base + c * _CHUNK, _CHUNK)],
            wsems[b])

    def wait_write(b):
        pltpu.make_async_copy(
            rows_v.at[b], out_hbm.at[pl.ds(base, _CHUNK)], wsems[b]).wait()

    def turn(c, rot, wwait, prefetch):
        if prefetch:
            b2 = (rot + _DIST) % _NBUF
            if wwait:
                wait_write(b2)
            start_gather(c + _DIST, b2)
        b = rot % _NBUF
        wait_gather(b)
        start_write(c, b)

    for c in range(_DIST):
        start_gather(c, c)

    head = _NBUF - _DIST
    for c in range(head):
        turn(c, c, wwait=False, prefetch=True)

    n_tail = _DIST
    steady = n_chunks - head - n_tail
    n_groups = steady // _NBUF
    rem = steady - n_groups * _NBUF

    def group(g, carry):
        c0 = head + g * _NBUF
        for j in range(_NBUF):
            turn(c0 + j, head + j, wwait=True, prefetch=True)
        return carry

    lax.fori_loop(0, n_groups, group, 0)

    for j in range(rem):
        c = head + n_groups * _NBUF + j
        turn(c, head + j, wwait=True, prefetch=True)

    for c in range(n_chunks - n_tail, n_chunks):
        turn(c, c, wwait=False, prefetch=False)

    for c in range(n_chunks - _NBUF, n_chunks):
        wait_write(c % _NBUF)


@jax.jit
def _gather(table, idx):
    n, d = idx.shape[0], table.shape[1]
    assert n % (_NW * _CHUNK) == 0
    bpw = n // _NW
    n_chunks = bpw // _CHUNK
    assert n_chunks >= 2 * _NBUF
    mesh = plsc.VectorSubcoreMesh(core_axis_name="c", subcore_axis_name="s")
    f = pl.kernel(
        functools.partial(_sc_body, n_chunks),
        out_type=jax.ShapeDtypeStruct((n, d), jnp.float32),
        mesh=mesh,
        scratch_types=(
            [pltpu.VMEM((bpw,), jnp.int32),
             pltpu.VMEM((_NBUF, _CHUNK, d), jnp.float32)]
            + [pltpu.SemaphoreType.DMA] * (2 * _NBUF)
        ),
    )
    return f(table, idx)


def kernel(token_ids, embedding):
    b, h = token_ids.shape
    d = embedding.shape[1]
    idx = token_ids.T.reshape(-1).astype(jnp.int32)
    out = _gather(embedding, idx)
    return out.reshape(h, b, d).transpose(1, 0, 2)
